# baseline (speedup 1.0000x reference)
import jax
import jax.numpy as jnp
from jax import lax
from jax.experimental import pallas as pl
from jax.experimental.pallas import tpu as pltpu
from jax.experimental.pallas import tpu_sc as plsc

B = 16384
D = 64
TPB = 8
L = 16
NC = 2
NS = 16
NW = NC * NS
B_PER_W = B // NW
CHUNK = 16
N_CHUNKS = B_PER_W // CHUNK
NBUF = 2


def _fire(tid16, tab_hbm, buf_ref, sem):
    for k in range(L):
        pltpu.async_copy(tab_hbm.at[tid16[k]], buf_ref.at[k], sem)


def _drain(tab_hbm, buf_ref, sem):
    pltpu.make_async_copy(tab_hbm.at[pl.ds(0, CHUNK)], buf_ref, sem).wait()


def _extract_body(tid_hbm, off_hbm, tab_hbm, emb_hbm,
                  tid_v, off_v, tb_v, ext_v, sem):
    wid = lax.axis_index("s") * NC + lax.axis_index("c")

    pltpu.sync_copy(tid_hbm.at[wid], tid_v)
    pltpu.sync_copy(off_hbm.at[wid], off_v)

    _fire(tid_v[pl.ds(0, L)], tab_hbm, tb_v.at[0], sem)

    lane = jnp.arange(L, dtype=jnp.int32)

    def chunk_body(j, _):
        buf = j % NBUF

        @pl.when(j + 1 < N_CHUNKS)
        def _():
            _fire(tid_v[pl.ds((j + 1) * CHUNK, L)], tab_hbm,
                  tb_v.at[(j + 1) % NBUF], sem)

        _drain(tab_hbm, tb_v.at[buf], sem)

        bufv = lane * 0 + buf
        slot = lane
        off = off_v[pl.ds(j * CHUNK, L)]
        zero = lane * 0
        for d in range(D):
            col = zero + d
            vals = plsc.load_gather(tb_v, [bufv, slot, off, col])
            ext_v[d, pl.ds(j * CHUNK, L)] = vals
        return 0

    lax.fori_loop(0, N_CHUNKS, chunk_body, 0)

    pltpu.sync_copy(ext_v, emb_hbm.at[:, pl.ds(wid * B_PER_W, B_PER_W)])


def _dot_body(tid_hbm, off_hbm, tab_hbm, emb_hbm, out_hbm,
              tid_v, off_v, tb_v, ucols_v, out_v, sem, usem):
    wid = lax.axis_index("s") * NC + lax.axis_index("c")

    pltpu.sync_copy(tid_hbm.at[wid], tid_v)
    pltpu.sync_copy(off_hbm.at[wid], off_v)
    ucp = pltpu.async_copy(
        emb_hbm.at[:, pl.ds(wid * B_PER_W, B_PER_W)], ucols_v, usem)

    _fire(tid_v[pl.ds(0, L)], tab_hbm, tb_v.at[0], sem)
    ucp.wait()

    lane = jnp.arange(L, dtype=jnp.int32)

    def chunk_body(j, _):
        buf = j % NBUF

        @pl.when(j + 1 < N_CHUNKS)
        def _():
            _fire(tid_v[pl.ds((j + 1) * CHUNK, L)], tab_hbm,
                  tb_v.at[(j + 1) % NBUF], sem)

        _drain(tab_hbm, tb_v.at[buf], sem)

        bufv = lane * 0 + buf
        slot = lane
        base = j * CHUNK
        off = off_v[pl.ds(base, L)]
        zero = lane * 0
        acc = (ucols_v[0, pl.ds(base, L)]
               * plsc.load_gather(tb_v, [bufv, slot, off, zero]))
        for d in range(1, D):
            col = zero + d
            acc = acc + (ucols_v[d, pl.ds(base, L)]
                         * plsc.load_gather(tb_v, [bufv, slot, off, col]))
        out_v[pl.ds(base, L)] = acc
        return 0

    lax.fori_loop(0, N_CHUNKS, chunk_body, 0)

    pltpu.sync_copy(out_v, out_hbm.at[pl.ds(wid * B_PER_W, B_PER_W)])


@jax.jit
def _run(user, item, user_table, item_table):
    mesh = plsc.VectorSubcoreMesh(core_axis_name="c", subcore_axis_name="s")
    params = pltpu.CompilerParams(needs_layout_passes=False)
    extract_k = pl.kernel(
        _extract_body,
        out_type=jax.ShapeDtypeStruct((D, B), jnp.float32),
        mesh=mesh,
        scratch_types=[
            pltpu.VMEM((B_PER_W,), jnp.int32),
            pltpu.VMEM((B_PER_W,), jnp.int32),
            pltpu.VMEM((NBUF, CHUNK, TPB, D), jnp.float32),
            pltpu.VMEM((D, B_PER_W), jnp.float32),
            pltpu.SemaphoreType.DMA,
        ],
        compiler_params=params,
    )
    dot_k = pl.kernel(
        _dot_body,
        out_type=jax.ShapeDtypeStruct((B,), jnp.float32),
        mesh=mesh,
        scratch_types=[
            pltpu.VMEM((B_PER_W,), jnp.int32),
            pltpu.VMEM((B_PER_W,), jnp.int32),
            pltpu.VMEM((NBUF, CHUNK, TPB, D), jnp.float32),
            pltpu.VMEM((D, B_PER_W), jnp.float32),
            pltpu.VMEM((B_PER_W,), jnp.float32),
            pltpu.SemaphoreType.DMA,
            pltpu.SemaphoreType.DMA,
        ],
        compiler_params=params,
    )
    u = user.astype(jnp.int32)
    i = item.astype(jnp.int32)
    ut3 = user_table.reshape(1000000 // TPB, TPB, D)
    it3 = item_table.reshape(1000000 // TPB, TPB, D)
    uemb = extract_k((u >> 3).reshape(NW, B_PER_W),
                     (u & 7).reshape(NW, B_PER_W), ut3)
    return dot_k((i >> 3).reshape(NW, B_PER_W),
                 (i & 7).reshape(NW, B_PER_W), it3, uemb)


def kernel(user, item, user_table, item_table):
    return _run(user, item, user_table, item_table)


# speedup vs baseline: 1.0377x; 1.0377x over previous
import jax
import jax.numpy as jnp
from jax import lax
from jax.experimental import pallas as pl
from jax.experimental.pallas import tpu as pltpu
from jax.experimental.pallas import tpu_sc as plsc

B = 16384
D = 64
TPB = 8
L = 16
NC = 2
NS = 16
NW = NC * NS
B_PER_W = B // NW
CHUNK = 16
N_CHUNKS = B_PER_W // CHUNK
NBUF = 2


def _sc_body(utid_hbm, itid_hbm, uoff_hbm, ioff_hbm,
             utab_hbm, itab_hbm, out_hbm,
             utid_v, itid_v, uoff_v, ioff_v, ub_v, ib_v, out_v, usem, isem):
    wid = lax.axis_index("s") * NC + lax.axis_index("c")

    pltpu.sync_copy(utid_hbm.at[wid], utid_v)
    pltpu.sync_copy(itid_hbm.at[wid], itid_v)
    pltpu.sync_copy(uoff_hbm.at[wid], uoff_v)
    pltpu.sync_copy(ioff_hbm.at[wid], ioff_v)

    def fire(j, buf):
        ut16 = utid_v[pl.ds(j * CHUNK, L)]
        it16 = itid_v[pl.ds(j * CHUNK, L)]
        for k in range(L):
            pltpu.async_copy(utab_hbm.at[ut16[k]], ub_v.at[buf, k], usem)
            pltpu.async_copy(itab_hbm.at[it16[k]], ib_v.at[buf, k], isem)

    def drain(buf):
        pltpu.make_async_copy(
            utab_hbm.at[pl.ds(0, CHUNK)], ub_v.at[buf], usem).wait()
        pltpu.make_async_copy(
            itab_hbm.at[pl.ds(0, CHUNK)], ib_v.at[buf], isem).wait()

    fire(0, 0)

    lane = jnp.arange(L, dtype=jnp.int32)

    def chunk_body(j, _):
        buf = j % NBUF

        @pl.when(j + 1 < N_CHUNKS)
        def _():
            fire(j + 1, (j + 1) % NBUF)

        drain(buf)

        bufv = lane * 0 + buf
        base = j * CHUNK
        uoff = uoff_v[pl.ds(base, L)]
        ioff = ioff_v[pl.ds(base, L)]
        zero = lane * 0
        acc = (plsc.load_gather(ub_v, [bufv, lane, uoff, zero])
               * plsc.load_gather(ib_v, [bufv, lane, ioff, zero]))
        for d in range(1, D):
            col = zero + d
            acc = acc + (plsc.load_gather(ub_v, [bufv, lane, uoff, col])
                         * plsc.load_gather(ib_v, [bufv, lane, ioff, col]))
        out_v[pl.ds(base, L)] = acc
        return 0

    lax.fori_loop(0, N_CHUNKS, chunk_body, 0)

    pltpu.sync_copy(out_v, out_hbm.at[pl.ds(wid * B_PER_W, B_PER_W)])


@jax.jit
def _run(user, item, user_table, item_table):
    mesh = plsc.VectorSubcoreMesh(core_axis_name="c", subcore_axis_name="s")
    kern = pl.kernel(
        _sc_body,
        out_type=jax.ShapeDtypeStruct((B,), jnp.float32),
        mesh=mesh,
        scratch_types=[
            pltpu.VMEM((B_PER_W,), jnp.int32),
            pltpu.VMEM((B_PER_W,), jnp.int32),
            pltpu.VMEM((B_PER_W,), jnp.int32),
            pltpu.VMEM((B_PER_W,), jnp.int32),
            pltpu.VMEM((NBUF, CHUNK, TPB, D), jnp.float32),
            pltpu.VMEM((NBUF, CHUNK, TPB, D), jnp.float32),
            pltpu.VMEM((B_PER_W,), jnp.float32),
            pltpu.SemaphoreType.DMA,
            pltpu.SemaphoreType.DMA,
        ],
        compiler_params=pltpu.CompilerParams(needs_layout_passes=False),
    )
    u = user.astype(jnp.int32)
    i = item.astype(jnp.int32)
    return kern(
        (u >> 3).reshape(NW, B_PER_W),
        (i >> 3).reshape(NW, B_PER_W),
        (u & 7).reshape(NW, B_PER_W),
        (i & 7).reshape(NW, B_PER_W),
        user_table.reshape(1000000 // TPB, TPB, D),
        item_table.reshape(1000000 // TPB, TPB, D),
    )


def kernel(user, item, user_table, item_table):
    return _run(user, item, user_table, item_table)
